# Initial kernel scaffold; baseline (speedup 1.0000x reference)
import jax
import jax.numpy as jnp
from jax.experimental import pallas as pl


def kernel(gen_emb, node_emb, rel_mat, src_pos, dst_pos, src_neg1, dst_neg1, src_neg2, dst_neg2):
    raise NotImplementedError("write your pallas kernel here")


# speedup vs baseline: 5.2124x; 5.2124x over previous
import functools

import jax
import jax.numpy as jnp
from jax import lax
from jax.experimental import pallas as pl
from jax.experimental.pallas import tpu as pltpu
from jax.experimental.pallas import tpu_sc as plsc

N = 50000
D = 64
R = 3
E = 65536

NC = 2
NS = 16
NW = NC * NS
EPW = E // NW
B = 128
NCH = EPW // B

BLK = 2000


def _combo_body(nemb_ref, rmat_ref, out_ref):
    out_ref[:, 0:D] = jnp.dot(nemb_ref[...], rmat_ref[0],
                              preferred_element_type=jnp.float32)
    out_ref[:, D:2 * D] = nemb_ref[...]


def _compute_combo(node_emb, rel_mat):
    return pl.pallas_call(
        _combo_body,
        grid=(R, N // BLK),
        in_specs=[
            pl.BlockSpec((BLK, D), lambda r, n: (n, 0)),
            pl.BlockSpec((1, D, D), lambda r, n: (r, 0, 0)),
        ],
        out_specs=pl.BlockSpec((BLK, 2 * D), lambda r, n: (r * (N // BLK) + n, 0)),
        out_shape=jax.ShapeDtypeStruct((R * N, 2 * D), jnp.float32),
    )(node_emb, rel_mat)


def _shuffle(a, idx):
    dnums = lax.GatherDimensionNumbers(
        offset_dims=(), collapsed_slice_dims=(0,), start_index_map=(0,))
    return lax.gather(a, idx[:, None], dnums, (1,),
                      mode=lax.GatherScatterMode.PROMISE_IN_BOUNDS)


def _sc_body(c_hbm, gen_hbm, sp, dp, sn1, dn1, sn2,
             out0, out1, out2, si_v, di_v, a_v, b_v, g_v, sc_v, sem):
    wid = lax.axis_index("s") * NC + lax.axis_index("c")
    lanes = lax.iota(jnp.int32, 16)
    zero16 = jnp.zeros((16,), jnp.float32)

    tasks = (
        (sp, dp, out0, True),
        (sn1, dn1, out1, True),
        (sn2, None, out2, False),
    )
    for src_hbm, dst_hbm, out_hbm, dst_is_gather in tasks:
        for r in range(R):
            roff = r * N

            def chunk_body(c, carry, src_hbm=src_hbm, dst_hbm=dst_hbm,
                           out_hbm=out_hbm, dst_is_gather=dst_is_gather,
                           roff=roff, r=r):
                base = r * E + wid * EPW + c * B
                pltpu.sync_copy(src_hbm.at[pl.ds(base, B)], si_v)
                for i in range(B // 16):
                    si_v[pl.ds(i * 16, 16)] = si_v[pl.ds(i * 16, 16)] + roff
                h1 = pltpu.async_copy(c_hbm.at[si_v], a_v, sem)
                if dst_is_gather:
                    pltpu.sync_copy(dst_hbm.at[pl.ds(base, B)], di_v)
                    for i in range(B // 16):
                        di_v[pl.ds(i * 16, 16)] = di_v[pl.ds(i * 16, 16)] + roff
                    h2 = pltpu.async_copy(c_hbm.at[di_v], b_v, sem)
                else:
                    h2 = pltpu.async_copy(gen_hbm.at[pl.ds(base, B), :],
                                          g_v, sem)
                h1.wait()
                h2.wait()

                def grp(g, carry2, dst_is_gather=dst_is_gather):
                    svec = zero16
                    for j in range(16):
                        e = g * 16 + j
                        if dst_is_gather:
                            acc = (a_v[e, pl.ds(0, 16)]
                                   * b_v[e, pl.ds(D, 16)])
                            for k in range(1, 4):
                                acc = acc + (a_v[e, pl.ds(k * 16, 16)]
                                             * b_v[e, pl.ds(D + k * 16, 16)])
                        else:
                            acc = (a_v[e, pl.ds(0, 16)]
                                   * g_v[e, pl.ds(0, 16)])
                            for k in range(1, 4):
                                acc = acc + (a_v[e, pl.ds(k * 16, 16)]
                                             * g_v[e, pl.ds(k * 16, 16)])
                        for dist in (1, 2, 4, 8):
                            acc = acc + _shuffle(acc, lanes ^ dist)
                        svec = svec + jnp.where(lanes == j, acc, zero16)
                    sc_v[pl.ds(g * 16, 16)] = svec
                    return carry2

                lax.fori_loop(0, B // 16, grp, 0)
                pltpu.sync_copy(sc_v, out_hbm.at[pl.ds(base, B)])
                return carry

            lax.fori_loop(0, NCH, chunk_body, 0)


_sc_kernel = functools.partial(
    pl.kernel,
    out_type=(
        jax.ShapeDtypeStruct((R * E,), jnp.float32),
        jax.ShapeDtypeStruct((R * E,), jnp.float32),
        jax.ShapeDtypeStruct((R * E,), jnp.float32),
    ),
    mesh=plsc.VectorSubcoreMesh(core_axis_name="c", subcore_axis_name="s"),
    scratch_types=[
        pltpu.VMEM((B,), jnp.int32),
        pltpu.VMEM((B,), jnp.int32),
        pltpu.VMEM((B, 2 * D), jnp.float32),
        pltpu.VMEM((B, 2 * D), jnp.float32),
        pltpu.VMEM((B, D), jnp.float32),
        pltpu.VMEM((B,), jnp.float32),
        pltpu.SemaphoreType.DMA,
    ],
)(_sc_body)


def kernel(gen_emb, node_emb, rel_mat, src_pos, dst_pos,
           src_neg1, dst_neg1, src_neg2, dst_neg2):
    combo = _compute_combo(node_emb, rel_mat)
    out0, out1, out2 = _sc_kernel(
        combo, gen_emb.reshape(R * E, D),
        src_pos.reshape(-1), dst_pos.reshape(-1),
        src_neg1.reshape(-1), dst_neg1.reshape(-1),
        src_neg2.reshape(-1),
    )
    return (out0, out1, out2)


# speedup vs baseline: 6.9797x; 1.3391x over previous
import functools

import jax
import jax.numpy as jnp
from jax import lax
from jax.experimental import pallas as pl
from jax.experimental.pallas import tpu as pltpu
from jax.experimental.pallas import tpu_sc as plsc

N = 50000
D = 64
R = 3
E = 65536

NC = 2
NS = 16
NW = NC * NS
EPW = E // NW
B = 128
HB = B // 2
NCH = EPW // B

BLK = 2000


def _combo_body(nemb_ref, rmat_ref, out_ref):
    out_ref[:, 0:D] = jnp.dot(nemb_ref[...], rmat_ref[0],
                              preferred_element_type=jnp.float32)
    out_ref[:, D:2 * D] = nemb_ref[...]


def _compute_combo(node_emb, rel_mat):
    return pl.pallas_call(
        _combo_body,
        grid=(R, N // BLK),
        in_specs=[
            pl.BlockSpec((BLK, D), lambda r, n: (n, 0)),
            pl.BlockSpec((1, D, D), lambda r, n: (r, 0, 0)),
        ],
        out_specs=pl.BlockSpec((BLK, 2 * D), lambda r, n: (r * (N // BLK) + n, 0)),
        out_shape=jax.ShapeDtypeStruct((R * N, 2 * D), jnp.float32),
    )(node_emb, rel_mat)


def _shuffle(a, idx):
    dnums = lax.GatherDimensionNumbers(
        offset_dims=(), collapsed_slice_dims=(0,), start_index_map=(0,))
    return lax.gather(a, idx[:, None], dnums, (1,),
                      mode=lax.GatherScatterMode.PROMISE_IN_BOUNDS)


def _sc_body(c_hbm, gen_hbm, sp, dp, sn1, dn1, sn2,
             out0, out1, out2,
             si_all, di_all, a_bufs, b_bufs, sc_all, sem0, sem1):
    wid = lax.axis_index("s") * NC + lax.axis_index("c")
    lanes = lax.iota(jnp.int32, 16)
    zero16 = jnp.zeros((16,), jnp.float32)
    sems = (sem0, sem1)

    tasks = (
        (sp, dp, out0, True),
        (sn1, dn1, out1, True),
        (sn2, None, out2, False),
    )
    for src_hbm, dst_hbm, out_hbm, dst_is_gather in tasks:
        for r in range(R):
            roff = r * N
            base_w = r * E + wid * EPW
            gbase = base_w // 2

            pltpu.sync_copy(src_hbm.at[pl.ds(base_w, EPW)], si_all)
            if dst_is_gather:
                pltpu.sync_copy(dst_hbm.at[pl.ds(base_w, EPW)], di_all)

            def adj(i, carry, dst_is_gather=dst_is_gather, roff=roff):
                si_all[pl.ds(i * 16, 16)] = si_all[pl.ds(i * 16, 16)] + roff
                if dst_is_gather:
                    di_all[pl.ds(i * 16, 16)] = di_all[pl.ds(i * 16, 16)] + roff
                return carry

            lax.fori_loop(0, EPW // 16, adj, 0)

            def fire(c, p, dst_is_gather=dst_is_gather, gbase=gbase):
                pltpu.async_copy(c_hbm.at[si_all.at[pl.ds(c * B, B)]],
                                 a_bufs.at[p], sems[p])
                if dst_is_gather:
                    pltpu.async_copy(c_hbm.at[di_all.at[pl.ds(c * B, B)]],
                                     b_bufs.at[p], sems[p])
                else:
                    goff_rows = pl.multiple_of(gbase + c * HB, 8)
                    pltpu.async_copy(
                        gen_hbm.at[pl.ds(goff_rows, HB), :],
                        b_bufs.at[p].at[pl.ds(0, HB), :], sems[p])

            def drain(p, dst_is_gather=dst_is_gather):
                pltpu.make_async_copy(c_hbm.at[si_all.at[pl.ds(0, B)]],
                                      a_bufs.at[p], sems[p]).wait()
                if dst_is_gather:
                    pltpu.make_async_copy(c_hbm.at[di_all.at[pl.ds(0, B)]],
                                          b_bufs.at[p], sems[p]).wait()
                else:
                    pltpu.make_async_copy(
                        gen_hbm.at[pl.ds(0, HB), :],
                        b_bufs.at[p].at[pl.ds(0, HB), :], sems[p]).wait()

            fire(0, 0)
            fire(1, 1)

            def chunk_body(c, carry, dst_is_gather=dst_is_gather,
                           fire=fire, drain=drain):
                par = lax.rem(c, 2)

                @pl.when(par == 0)
                def _():
                    drain(0)

                @pl.when(par == 1)
                def _():
                    drain(1)

                def grp(g, carry2, dst_is_gather=dst_is_gather):
                    svec = zero16
                    for j in range(16):
                        e = g * 16 + j
                        if dst_is_gather:
                            acc = (a_bufs[par, e, pl.ds(0, 16)]
                                   * b_bufs[par, e, pl.ds(D, 16)])
                            for k in range(1, 4):
                                acc = acc + (
                                    a_bufs[par, e, pl.ds(k * 16, 16)]
                                    * b_bufs[par, e, pl.ds(D + k * 16, 16)])
                        else:
                            gr = g * 8 + j // 2
                            goff = (j % 2) * D
                            acc = (a_bufs[par, e, pl.ds(0, 16)]
                                   * b_bufs[par, gr, pl.ds(goff, 16)])
                            for k in range(1, 4):
                                acc = acc + (
                                    a_bufs[par, e, pl.ds(k * 16, 16)]
                                    * b_bufs[par, gr, pl.ds(goff + k * 16, 16)])
                        for dist in (1, 2, 4, 8):
                            acc = acc + _shuffle(acc, lanes ^ dist)
                        svec = svec + jnp.where(lanes == j, acc, zero16)
                    sc_all[pl.ds(c * B + g * 16, 16)] = svec
                    return carry2

                lax.fori_loop(0, B // 16, grp, 0)

                @pl.when(jnp.logical_and(par == 0, c + 2 < NCH))
                def _():
                    fire(c + 2, 0)

                @pl.when(jnp.logical_and(par == 1, c + 2 < NCH))
                def _():
                    fire(c + 2, 1)

                return carry

            lax.fori_loop(0, NCH, chunk_body, 0)
            pltpu.sync_copy(sc_all, out_hbm.at[pl.ds(base_w, EPW)])


_sc_kernel = functools.partial(
    pl.kernel,
    out_type=(
        jax.ShapeDtypeStruct((R * E,), jnp.float32),
        jax.ShapeDtypeStruct((R * E,), jnp.float32),
        jax.ShapeDtypeStruct((R * E,), jnp.float32),
    ),
    mesh=plsc.VectorSubcoreMesh(core_axis_name="c", subcore_axis_name="s"),
    scratch_types=[
        pltpu.VMEM((EPW,), jnp.int32),
        pltpu.VMEM((EPW,), jnp.int32),
        pltpu.VMEM((2, B, 2 * D), jnp.float32),
        pltpu.VMEM((2, B, 2 * D), jnp.float32),
        pltpu.VMEM((EPW,), jnp.float32),
        pltpu.SemaphoreType.DMA,
        pltpu.SemaphoreType.DMA,
    ],
)(_sc_body)


def kernel(gen_emb, node_emb, rel_mat, src_pos, dst_pos,
           src_neg1, dst_neg1, src_neg2, dst_neg2):
    combo = _compute_combo(node_emb, rel_mat)
    out0, out1, out2 = _sc_kernel(
        combo, gen_emb.reshape(R * E // 2, 2 * D),
        src_pos.reshape(-1), dst_pos.reshape(-1),
        src_neg1.reshape(-1), dst_neg1.reshape(-1),
        src_neg2.reshape(-1),
    )
    return (out0, out1, out2)


# speedup vs baseline: 7.8979x; 1.1316x over previous
import functools

import jax
import jax.numpy as jnp
from jax import lax
from jax.experimental import pallas as pl
from jax.experimental.pallas import tpu as pltpu
from jax.experimental.pallas import tpu_sc as plsc

N = 50000
D = 64
R = 3
E = 65536

NC = 2
NS = 16
NW = NC * NS
EPW = E // NW
B = 128
HB = B // 2
NCH = EPW // B

BLK = 2000


def _combo_body(nemb_ref, rmat_ref, out_ref):
    out_ref[:, 0:D] = jnp.dot(nemb_ref[...], rmat_ref[0],
                              preferred_element_type=jnp.float32)
    out_ref[:, D:2 * D] = nemb_ref[...]


def _compute_combo(node_emb, rel_mat):
    return pl.pallas_call(
        _combo_body,
        grid=(R, N // BLK),
        in_specs=[
            pl.BlockSpec((BLK, D), lambda r, n: (n, 0)),
            pl.BlockSpec((1, D, D), lambda r, n: (r, 0, 0)),
        ],
        out_specs=pl.BlockSpec((BLK, 2 * D), lambda r, n: (r * (N // BLK) + n, 0)),
        out_shape=jax.ShapeDtypeStruct((R * N, 2 * D), jnp.float32),
    )(node_emb, rel_mat)


def _shuffle(a, idx):
    dnums = lax.GatherDimensionNumbers(
        offset_dims=(), collapsed_slice_dims=(0,), start_index_map=(0,))
    return lax.gather(a, idx[:, None], dnums, (1,),
                      mode=lax.GatherScatterMode.PROMISE_IN_BOUNDS)


def _sc_body(c_hbm, gen_hbm, sp, dp, sn1, dn1, sn2,
             out0, out1, out2,
             si_all, di_all, a_bufs, b_bufs, g_bufs, sc_all, sem0, sem1):
    wid = lax.axis_index("s") * NC + lax.axis_index("c")
    lanes = lax.iota(jnp.int32, 16)
    zero16 = jnp.zeros((16,), jnp.float32)
    sems = (sem0, sem1)

    tasks = (
        (sp, dp, out0, True),
        (sn1, dn1, out1, True),
        (sn2, None, out2, False),
    )
    for src_hbm, dst_hbm, out_hbm, dst_is_gather in tasks:
        for r in range(R):
            roff = r * N
            base_w = r * E + wid * EPW

            pltpu.sync_copy(src_hbm.at[pl.ds(base_w, EPW)], si_all)
            if dst_is_gather:
                pltpu.sync_copy(dst_hbm.at[pl.ds(base_w, EPW)], di_all)

            def adj(i, carry, dst_is_gather=dst_is_gather, roff=roff):
                si_all[pl.ds(i * 16, 16)] = si_all[pl.ds(i * 16, 16)] + roff
                if dst_is_gather:
                    di_all[pl.ds(i * 16, 16)] = di_all[pl.ds(i * 16, 16)] + roff
                return carry

            lax.fori_loop(0, EPW // 16, adj, 0)

            def fire(c, p, dst_is_gather=dst_is_gather, base_w=base_w):
                pltpu.async_copy(c_hbm.at[si_all.at[pl.ds(c * B, B)]],
                                 a_bufs.at[p], sems[p])
                if dst_is_gather:
                    pltpu.async_copy(c_hbm.at[di_all.at[pl.ds(c * B, B)]],
                                     b_bufs.at[p], sems[p])
                else:
                    goff_rows = pl.multiple_of(base_w + c * B, 8)
                    pltpu.async_copy(
                        gen_hbm.at[pl.ds(goff_rows, B), :],
                        g_bufs.at[p], sems[p])

            def drain(p, dst_is_gather=dst_is_gather):
                pltpu.make_async_copy(c_hbm.at[si_all.at[pl.ds(0, B)]],
                                      a_bufs.at[p], sems[p]).wait()
                if dst_is_gather:
                    pltpu.make_async_copy(c_hbm.at[di_all.at[pl.ds(0, B)]],
                                          b_bufs.at[p], sems[p]).wait()
                else:
                    pltpu.make_async_copy(
                        gen_hbm.at[pl.ds(0, B), :],
                        g_bufs.at[p], sems[p]).wait()

            fire(0, 0)
            fire(1, 1)

            def chunk_body(c, carry, dst_is_gather=dst_is_gather,
                           fire=fire, drain=drain):
                par = lax.rem(c, 2)

                @pl.when(par == 0)
                def _():
                    drain(0)

                @pl.when(par == 1)
                def _():
                    drain(1)

                def grp(g, carry2, dst_is_gather=dst_is_gather):
                    svec = zero16
                    for j in range(16):
                        e = g * 16 + j
                        if dst_is_gather:
                            acc = (a_bufs[par, e, pl.ds(0, 16)]
                                   * b_bufs[par, e, pl.ds(D, 16)])
                            for k in range(1, 4):
                                acc = acc + (
                                    a_bufs[par, e, pl.ds(k * 16, 16)]
                                    * b_bufs[par, e, pl.ds(D + k * 16, 16)])
                        else:
                            acc = (a_bufs[par, e, pl.ds(0, 16)]
                                   * g_bufs[par, e, pl.ds(0, 16)])
                            for k in range(1, 4):
                                acc = acc + (
                                    a_bufs[par, e, pl.ds(k * 16, 16)]
                                    * g_bufs[par, e, pl.ds(k * 16, 16)])
                        for dist in (1, 2, 4, 8):
                            acc = acc + _shuffle(acc, lanes ^ dist)
                        svec = svec + jnp.where(lanes == j, acc, zero16)
                    sc_all[pl.ds(c * B + g * 16, 16)] = svec
                    return carry2

                lax.fori_loop(0, B // 16, grp, 0)

                @pl.when(jnp.logical_and(par == 0, c + 2 < NCH))
                def _():
                    fire(c + 2, 0)

                @pl.when(jnp.logical_and(par == 1, c + 2 < NCH))
                def _():
                    fire(c + 2, 1)

                return carry

            lax.fori_loop(0, NCH, chunk_body, 0)
            pltpu.sync_copy(sc_all, out_hbm.at[pl.ds(base_w, EPW)])


_sc_kernel = functools.partial(
    pl.kernel,
    out_type=(
        jax.ShapeDtypeStruct((R * E,), jnp.float32),
        jax.ShapeDtypeStruct((R * E,), jnp.float32),
        jax.ShapeDtypeStruct((R * E,), jnp.float32),
    ),
    mesh=plsc.VectorSubcoreMesh(core_axis_name="c", subcore_axis_name="s"),
    scratch_types=[
        pltpu.VMEM((EPW,), jnp.int32),
        pltpu.VMEM((EPW,), jnp.int32),
        pltpu.VMEM((2, B, 2 * D), jnp.float32),
        pltpu.VMEM((2, B, 2 * D), jnp.float32),
        pltpu.VMEM((2, B, D), jnp.float32),
        pltpu.VMEM((EPW,), jnp.float32),
        pltpu.SemaphoreType.DMA,
        pltpu.SemaphoreType.DMA,
    ],
)(_sc_body)


def kernel(gen_emb, node_emb, rel_mat, src_pos, dst_pos,
           src_neg1, dst_neg1, src_neg2, dst_neg2):
    combo = _compute_combo(node_emb, rel_mat)
    out0, out1, out2 = _sc_kernel(
        combo, gen_emb.reshape(R * E, D),
        src_pos.reshape(-1), dst_pos.reshape(-1),
        src_neg1.reshape(-1), dst_neg1.reshape(-1),
        src_neg2.reshape(-1),
    )
    return (out0, out1, out2)
